# baseline (speedup 1.0000x reference)
import functools

import jax
import jax.numpy as jnp
from jax import lax
from jax.experimental import pallas as pl
from jax.experimental.pallas import tpu as pltpu
from jax.experimental.pallas import tpu_sc as plsc

F32 = jnp.float32
H = 64
N = 10000
E = 160000
NW = 32
C = 128
EPW = 5120
EP = NW * EPW
NCHUNK = EPW // C
ZBIG = 632
ZLAST = N - 15 * ZBIG
DZ = 5120
RB = 1000
NRB = N // RB
DN = (((1,), (1,)), ((), ()))
HI = lax.Precision.HIGHEST
NPAIR = NCHUNK // 2
NP0 = 13
NPAIR2 = 2 * NPAIR


def _mm(a, b):
    return lax.dot_general(a.astype(jnp.bfloat16), b.astype(jnp.bfloat16),
                           DN, preferred_element_type=F32)


def _proj_body(xc_ref, xm_ref, wci_ref, bci_ref, wmi_ref, bmi_ref,
               oc_ref, om_ref):
    oc_ref[...] = (_mm(xc_ref[...], wci_ref[...])
                   + bci_ref[...][None, :])
    om_ref[...] = (_mm(xm_ref[...], wmi_ref[...])
                   + bmi_ref[...][None, :])


def _tc_proj(xc, xm, wci, bci, wmi, bmi):
    return pl.pallas_call(
        _proj_body,
        out_shape=[jax.ShapeDtypeStruct((N, H), F32),
                   jax.ShapeDtypeStruct((N, H), F32)],
    )(xc, xm, wci, bci, wmi, bmi)


def _prep_body(ph, xs_ref, xd_ref, w_ref, as_ref, ad_ref,
               zt_ref, alt_ref, adt_ref):
    zs = _mm(xs_ref[...], w_ref[...])
    zd = _mm(xd_ref[...], w_ref[...])
    zt_ref[...] = zs
    av = as_ref[0]
    dv = ad_ref[0]
    alt_ref[...] = jnp.stack(
        [jnp.sum(zs[:, q * H:(q + 1) * H] * av[q][None, :], axis=1)
         for q in range(ph)], axis=1)
    adt_ref[...] = jnp.stack(
        [jnp.sum(zd[:, q * H:(q + 1) * H] * dv[q][None, :], axis=1)
         for q in range(ph)], axis=1)


def _tc_prep(hd, ph, xs, xd, w, a_s, a_d):
    din = xs.shape[1]
    np_ = hd // ph
    wp = ph * H
    zt, alt, adt = pl.pallas_call(
        functools.partial(_prep_body, ph),
        grid=(np_, NRB),
        in_specs=[
            pl.BlockSpec((RB, din), lambda p, i: (i, 0)),
            pl.BlockSpec((RB, din), lambda p, i: (i, 0)),
            pl.BlockSpec((wp, din), lambda p, i: (p, 0)),
            pl.BlockSpec((1, ph, H), lambda p, i: (p, 0, 0)),
            pl.BlockSpec((1, ph, H), lambda p, i: (p, 0, 0)),
        ],
        out_specs=[
            pl.BlockSpec((RB, wp), lambda p, i: (p * NRB + i, 0)),
            pl.BlockSpec((RB, ph), lambda p, i: (p * NRB + i, 0)),
            pl.BlockSpec((RB, ph), lambda p, i: (p * NRB + i, 0)),
        ],
        out_shape=[jax.ShapeDtypeStruct((np_ * N, wp), F32),
                   jax.ShapeDtypeStruct((np_ * N, ph), F32),
                   jax.ShapeDtypeStruct((np_ * N, ph), F32)],
    )(xs, xd, w, a_s.reshape(np_, ph, H), a_d.reshape(np_, ph, H))
    zpad = jnp.zeros((16,), F32)
    return (zt, jnp.concatenate([alt.reshape(-1), zpad]),
            jnp.concatenate([adt.reshape(-1), zpad]))


def _p1_body(ph, part_ref, den_ref, b_ref, nm_ref, st_ref):
    i = pl.program_id(1)
    agg = part_ref[0, 0] + part_ref[1, 0]
    dd = den_ref[0]
    bb = b_ref[0, 0]
    xs, news = [], []
    for q in range(ph):
        den_q = dd[:, 2 * q] + dd[:, 2 * q + 1]
        x = (agg[:, q * H:(q + 1) * H] / (den_q[:, None] + 1e-16)
             + bb[q * H:(q + 1) * H][None, :])
        xs.append(x[None])
        s1 = jnp.sum(x, axis=0)[None, :]
        s2 = jnp.sum(x * x, axis=0)[None, :]
        news.append(jnp.concatenate([s1, s2, jnp.zeros((6, H), F32)],
                                    axis=0)[None])
    nm_ref[...] = jnp.concatenate(xs, axis=0)
    new = jnp.concatenate(news, axis=0)

    @pl.when(i == 0)
    def _init():
        st_ref[...] = new

    @pl.when(i > 0)
    def _acc():
        st_ref[...] = st_ref[...] + new


def _tc_combine(hd, ph, part, den4, b):
    np_ = hd // ph
    wp = ph * H
    return pl.pallas_call(
        functools.partial(_p1_body, ph),
        grid=(np_, NRB),
        in_specs=[
            pl.BlockSpec((2, 1, RB, wp), lambda p, i: (0, p, i, 0)),
            pl.BlockSpec((1, RB, 2 * ph), lambda p, i: (p, i, 0)),
            pl.BlockSpec((1, 1, wp), lambda p, i: (p, 0, 0)),
        ],
        out_specs=[
            pl.BlockSpec((ph, RB, H), lambda p, i: (p, i, 0)),
            pl.BlockSpec((ph, 8, H), lambda p, i: (p, 0, 0)),
        ],
        out_shape=[jax.ShapeDtypeStruct((hd, N, H), F32),
                   jax.ShapeDtypeStruct((hd, 8, H), F32)],
    )(part, den4, b.reshape(np_, 1, wp))


def _p2_body(hd, emit_table, nm_ref, st_ref, g_ref, be_ref, w2_ref, a2_ref,
             zt2_ref, alt2_ref):
    acc = jnp.zeros((RB, H), F32)
    for h in range(hd):
        x = nm_ref[h]
        mu = st_ref[h, 0] * (1.0 / N)
        var = st_ref[h, 1] * (1.0 / N) - mu * mu
        xb = jnp.maximum(
            g_ref[h][None, :] * (x - mu[None, :])
            / jnp.sqrt(var + 1e-5)[None, :] + be_ref[h][None, :], 0.0)
        acc = acc + _mm(xb, w2_ref[h])
    alt2_ref[...] = jnp.sum(acc * a2_ref[...], axis=1).reshape(RB, 1)
    if emit_table:
        zt2_ref[...] = acc


def _tc_bn_prep2(hd, emit_table, nm, st, g, be, w2r, a2):
    zt2, alt2 = pl.pallas_call(
        functools.partial(_p2_body, hd, emit_table),
        grid=(NRB,),
        in_specs=[
            pl.BlockSpec((hd, RB, H), lambda i: (0, i, 0)),
            pl.BlockSpec((hd, 8, H), lambda i: (0, 0, 0)),
            pl.BlockSpec((hd, H), lambda i: (0, 0)),
            pl.BlockSpec((hd, H), lambda i: (0, 0)),
            pl.BlockSpec((hd, H, H), lambda i: (0, 0, 0)),
            pl.BlockSpec((1, H), lambda i: (0, 0)),
        ],
        out_specs=[
            pl.BlockSpec((RB, H), lambda i: (i, 0)),
            pl.BlockSpec((RB, 1), lambda i: (i, 0)),
        ],
        out_shape=[jax.ShapeDtypeStruct((N, H), F32),
                   jax.ShapeDtypeStruct((N, 1), F32)],
    )(nm, st, g.reshape(hd, H), be.reshape(hd, H), w2r, a2)
    return zt2, alt2.reshape(-1)


def _final_body(part_ref, den_ref, b_ref, g_ref, be_ref,
                wc1_ref, bc1_ref, wc2_ref, bc2_ref, out_ref):
    agg = part_ref[0, 0] + part_ref[1, 0]
    den = den_ref[..., 0] + den_ref[..., 1]
    nc = agg / (den[:, None] + 1e-16) + b_ref[...][None, :]
    mu = jnp.mean(nc, axis=0)
    var = jnp.mean((nc - mu[None, :]) ** 2, axis=0)
    nc = jnp.maximum(
        g_ref[...][None, :] * (nc - mu[None, :])
        / jnp.sqrt(var + 1e-5)[None, :] + be_ref[...][None, :], 0.0)
    hh = jnp.maximum(
        _mm(nc, wc1_ref[...])
        + bc1_ref[...][None, :], 0.0)
    out_ref[...] = (_mm(hh, wc2_ref[...])
                    + bc2_ref[...][None, :])


def _tc_final(part, denT, b, g, be, wc1, bc1, wc2, bc2):
    return pl.pallas_call(
        _final_body,
        out_shape=jax.ShapeDtypeStruct((N, 2), F32),
    )(part, denT[:N], b, g, be, wc1, bc1, wc2, bc2)


def _sc_gat(hd, ph, sd, alsT, aldT, zsT, z2d, z1d):
    np_ = hd // ph
    wp = ph * H
    nz = -(-(hd * N + ph) // DZ)
    mden = nz * DZ
    mesh = plsc.VectorSubcoreMesh(core_axis_name="c", subcore_axis_name="s")

    @functools.partial(
        pl.kernel,
        out_type=[jax.ShapeDtypeStruct((2, np_, N, wp), F32),
                  jax.ShapeDtypeStruct((2 * mden,), F32)],
        mesh=mesh,
        scratch_types=[
            [pltpu.VMEM((2 * C,), jnp.int32) for _ in range(2)],
            [pltpu.VMEM((C,), jnp.int32) for _ in range(2)],
            [pltpu.VMEM((C,), jnp.int32) for _ in range(2)],
            [[pltpu.VMEM((C,), jnp.int32) for _ in range(ph)]
             for _ in range(2)],
            [[pltpu.VMEM((C,), jnp.int32) for _ in range(ph)]
             for _ in range(2)],
            [[pltpu.VMEM((C,), F32) for _ in range(ph)]
             for _ in range(2)],
            [[pltpu.VMEM((C,), F32) for _ in range(ph)]
             for _ in range(2)],
            [[pltpu.VMEM((C,), F32) for _ in range(ph)]
             for _ in range(2)],
            [pltpu.VMEM((C, wp), F32) for _ in range(2)],
            [pltpu.SemaphoreType.DMA for _ in range(2)],
            pltpu.VMEM_SHARED((N + 8, wp), F32),
            pltpu.VMEM_SHARED((mden,), F32),
        ],
        compiler_params=pltpu.CompilerParams(use_tc_tiling_on_sc=False),
    )
    def k(sd_hbm, als_hbm, ald_hbm, zst_hbm, z2d_hbm, z1d_hbm,
          part_hbm, den_hbm,
          sd_v, iz_v, d_v, ia_v, id_v, as_v, ad_v, ex_v, g_v, sem,
          agg_sh, den_sh):
        cid = lax.axis_index("c")
        sid = lax.axis_index("s")
        npair = jnp.where(cid == 0, NP0, NPAIR2 - NP0)
        wchunk = jnp.where(cid == 0, sid * (2 * NP0),
                           32 * NP0 + sid * (2 * (NPAIR2 - NP0)))

        @pl.when(sid < nz)
        def _zero_den():
            pltpu.sync_copy(z1d_hbm, den_sh.at[pl.ds(sid * DZ, DZ)])

        for p in range(np_):
            plsc.subcore_barrier()

            @pl.when(sid < 15)
            def _zero_big():
                pltpu.sync_copy(z2d_hbm,
                                agg_sh.at[pl.ds(sid * ZBIG, ZBIG), :])

            @pl.when(sid == 15)
            def _zero_last():
                pltpu.sync_copy(z2d_hbm.at[pl.ds(0, ZLAST + 8), :],
                                agg_sh.at[pl.ds(15 * ZBIG, ZLAST + 8), :])

            plsc.subcore_barrier()

            def issue(kb, b):
                base = (wchunk + kb) * (2 * C)
                pltpu.sync_copy(sd_hbm.at[pl.ds(base, 2 * C)], sd_v[b])
                for v in range(C // 16):
                    sl = pl.ds(v * 16, 16)
                    s16 = sd_v[b][sl] + (p * N)
                    d16 = sd_v[b][pl.ds(C + v * 16, 16)]
                    iz_v[b][sl] = s16
                    d_v[b][sl] = d16
                    dp = (d16 + (p * N)) * ph
                    sp = s16 * ph
                    for q in range(ph):
                        ia_v[b][q][sl] = sp + q
                        id_v[b][q][sl] = dp + q
                for q in range(ph):
                    pltpu.async_copy(als_hbm.at[ia_v[b][q]], as_v[b][q],
                                     sem[b])
                    pltpu.async_copy(ald_hbm.at[id_v[b][q]], ad_v[b][q],
                                     sem[b])
                pltpu.async_copy(zst_hbm.at[iz_v[b]], g_v[b], sem[b])

            def process(b):
                for q in range(ph):
                    pltpu.make_async_copy(als_hbm.at[ia_v[b][q]],
                                          as_v[b][q], sem[b]).wait()
                    pltpu.make_async_copy(ald_hbm.at[id_v[b][q]],
                                          ad_v[b][q], sem[b]).wait()
                pltpu.make_async_copy(zst_hbm.at[iz_v[b]], g_v[b],
                                      sem[b]).wait()
                for q in range(ph):
                    for v in range(C // 16):
                        sl = pl.ds(v * 16, 16)
                        x = as_v[b][q][sl] + ad_v[b][q][sl]
                        ex_v[b][q][sl] = jnp.exp(jnp.maximum(x, 0.2 * x))
                    pltpu.sync_copy(ex_v[b][q], den_sh.at[id_v[b][q]],
                                    add=True)

                def scale(g, c2):
                    exq = [ex_v[b][q][pl.ds(g * 16, 16)]
                           for q in range(ph)]
                    for l in range(16):
                        i = g * 16 + l
                        for q in range(ph):
                            sv = jnp.full((16,), exq[q][l], F32)
                            for j in range(H // 16):
                                slj = pl.ds((q * (H // 16) + j) * 16, 16)
                                g_v[b][i, slj] = g_v[b][i, slj] * sv
                    return c2

                lax.fori_loop(0, C // 16, scale, 0)
                pltpu.sync_copy(g_v[b], agg_sh.at[d_v[b]], add=True)

            issue(0, 0)

            def pair(pp, carry):
                k0 = 2 * pp
                issue(k0 + 1, 1)
                process(0)

                @pl.when(pp + 1 < npair)
                def _next():
                    issue(k0 + 2, 0)

                process(1)
                return carry

            lax.fori_loop(0, npair, pair, 0)
            plsc.subcore_barrier()

            @pl.when(sid < 15)
            def _wb_big():
                pltpu.sync_copy(
                    agg_sh.at[pl.ds(sid * ZBIG, ZBIG), :],
                    part_hbm.at[cid, p, pl.ds(sid * ZBIG, ZBIG), :])

            @pl.when(sid == 15)
            def _wb_last():
                pltpu.sync_copy(
                    agg_sh.at[pl.ds(15 * ZBIG, ZLAST), :],
                    part_hbm.at[cid, p, pl.ds(15 * ZBIG, ZLAST), :])

        plsc.subcore_barrier()

        @pl.when(sid == 0)
        def _write_den():
            pltpu.sync_copy(den_sh, den_hbm.at[pl.ds(cid * mden, mden)])

    part, den = k(sd, alsT, aldT, zsT, z2d, z1d)
    return part, den, mden


def kernel(x_card, x_merchant, edge_index_cm, edge_index_mc,
           W_ci, b_ci, W_mi, b_mi,
           W1_cm, as1_cm, ad1_cm, b1_cm, W1_mc, as1_mc, ad1_mc, b1_mc,
           g1_c, be1_c, g1_m, be1_m,
           W2_cm, as2_cm, ad2_cm, b2_cm, W2_mc, as2_mc, ad2_mc, b2_mc,
           g2_c, be2_c, g2_m, be2_m,
           Wc1, bc1, Wc2, bc2):
    ei_cm = edge_index_cm.astype(jnp.int32)
    ei_mc = edge_index_mc.astype(jnp.int32)
    pad_s = jnp.zeros((EP - E,), jnp.int32)
    pad_d = jnp.full((EP - E,), N, jnp.int32)

    def _pack(src, dst):
        s = jnp.concatenate([src, pad_s]).reshape(-1, C)
        d = jnp.concatenate([dst, pad_d]).reshape(-1, C)
        return jnp.concatenate([s, d], axis=1).reshape(-1)

    sd_cm = _pack(ei_cm[0], ei_cm[1])
    sd_mc = _pack(ei_mc[0], ei_mc[1])
    z2dw = jnp.zeros((ZBIG, 2 * H), F32)
    z2d1 = jnp.zeros((ZBIG, H), F32)
    z1d = jnp.zeros((DZ,), F32)
    w2r = W2_mc.reshape(H, 4, H).transpose(1, 0, 2)
    zp16 = jnp.zeros((16,), F32)

    xc, xm = _tc_proj(x_card, x_merchant, W_ci, b_ci, W_mi, b_mi)

    zsT_cm, alsT_cm, aldT_cm = _tc_prep(4, 2, xc, xm, W1_cm, as1_cm, ad1_cm)
    zsT_mc, alsT_mc, aldT_mc = _tc_prep(4, 2, xm, xc, W1_mc, as1_mc, ad1_mc)

    part_cm, den_cm, md1 = _sc_gat(4, 2, sd_cm, alsT_cm, aldT_cm, zsT_cm,
                                   z2dw, z1d)
    part_mc, den_mc, _ = _sc_gat(4, 2, sd_mc, alsT_mc, aldT_mc, zsT_mc,
                                 z2dw, z1d)

    def _den4(den):
        return jnp.stack([den[:4 * N], den[md1:md1 + 4 * N]],
                         axis=-1).reshape(2, N, 4)

    nm1, st_m = _tc_combine(4, 2, part_cm, _den4(den_cm), b1_cm)
    nc1, st_c = _tc_combine(4, 2, part_mc, _den4(den_mc), b1_mc)
    zsT2, als2 = _tc_bn_prep2(4, True, nm1, st_m, g1_m, be1_m, w2r, as2_mc)
    _, ald2 = _tc_bn_prep2(4, False, nc1, st_c, g1_c, be1_c, w2r, ad2_mc)
    als2 = jnp.concatenate([als2, zp16])
    ald2 = jnp.concatenate([ald2, zp16])

    part2, den2, md2 = _sc_gat(1, 1, sd_mc, als2, ald2, zsT2, z2d1, z1d)
    den2T = jnp.stack([den2[:N], den2[md2:md2 + N]], axis=-1)

    return _tc_final(part2, den2T, b2_mc, g2_c, be2_c, Wc1, bc1, Wc2, bc2)


# speedup vs baseline: 1.1363x; 1.1363x over previous
import functools

import jax
import jax.numpy as jnp
from jax import lax
from jax.experimental import pallas as pl
from jax.experimental.pallas import tpu as pltpu
from jax.experimental.pallas import tpu_sc as plsc

F32 = jnp.float32
H = 64
N = 10000
E = 160000
NW = 32
C = 128
EPW = 5120
EP = NW * EPW
NCHUNK = EPW // C
ZBIG = 632
ZLAST = N - 15 * ZBIG
DZ = 5120
RB = 1000
NRB = N // RB
DN = (((1,), (1,)), ((), ()))
HI = lax.Precision.HIGHEST
NPAIR = NCHUNK // 2
NP0 = 27
NPAIR2 = 2 * NPAIR


def _mm(a, b):
    return lax.dot_general(a.astype(jnp.bfloat16), b.astype(jnp.bfloat16),
                           DN, preferred_element_type=F32)


def _proj_body(xc_ref, xm_ref, wci_ref, bci_ref, wmi_ref, bmi_ref,
               oc_ref, om_ref):
    oc_ref[...] = (_mm(xc_ref[...], wci_ref[...])
                   + bci_ref[...][None, :])
    om_ref[...] = (_mm(xm_ref[...], wmi_ref[...])
                   + bmi_ref[...][None, :])


def _tc_proj(xc, xm, wci, bci, wmi, bmi):
    return pl.pallas_call(
        _proj_body,
        out_shape=[jax.ShapeDtypeStruct((N, H), F32),
                   jax.ShapeDtypeStruct((N, H), F32)],
    )(xc, xm, wci, bci, wmi, bmi)


def _prep_body(ph, xs_ref, xd_ref, w_ref, as_ref, ad_ref,
               zt_ref, alt_ref, adt_ref):
    zs = _mm(xs_ref[...], w_ref[...])
    zd = _mm(xd_ref[...], w_ref[...])
    zt_ref[...] = zs
    av = as_ref[0]
    dv = ad_ref[0]
    alt_ref[...] = jnp.stack(
        [jnp.sum(zs[:, q * H:(q + 1) * H] * av[q][None, :], axis=1)
         for q in range(ph)], axis=1)
    adt_ref[...] = jnp.stack(
        [jnp.sum(zd[:, q * H:(q + 1) * H] * dv[q][None, :], axis=1)
         for q in range(ph)], axis=1)


def _tc_prep(hd, ph, xs, xd, w, a_s, a_d):
    din = xs.shape[1]
    np_ = hd // ph
    wp = ph * H
    zt, alt, adt = pl.pallas_call(
        functools.partial(_prep_body, ph),
        grid=(np_, NRB),
        in_specs=[
            pl.BlockSpec((RB, din), lambda p, i: (i, 0)),
            pl.BlockSpec((RB, din), lambda p, i: (i, 0)),
            pl.BlockSpec((wp, din), lambda p, i: (p, 0)),
            pl.BlockSpec((1, ph, H), lambda p, i: (p, 0, 0)),
            pl.BlockSpec((1, ph, H), lambda p, i: (p, 0, 0)),
        ],
        out_specs=[
            pl.BlockSpec((RB, wp), lambda p, i: (p * NRB + i, 0)),
            pl.BlockSpec((RB, ph), lambda p, i: (p * NRB + i, 0)),
            pl.BlockSpec((RB, ph), lambda p, i: (p * NRB + i, 0)),
        ],
        out_shape=[jax.ShapeDtypeStruct((np_ * N, wp), F32),
                   jax.ShapeDtypeStruct((np_ * N, ph), F32),
                   jax.ShapeDtypeStruct((np_ * N, ph), F32)],
    )(xs, xd, w, a_s.reshape(np_, ph, H), a_d.reshape(np_, ph, H))
    zpad = jnp.zeros((16,), F32)
    return (zt, jnp.concatenate([alt.reshape(-1), zpad]),
            jnp.concatenate([adt.reshape(-1), zpad]))


def _p1_body(ph, part_ref, den_ref, b_ref, nm_ref, st_ref):
    i = pl.program_id(1)
    agg = part_ref[0, 0] + part_ref[1, 0]
    dd = den_ref[0]
    bb = b_ref[0, 0]
    xs, news = [], []
    for q in range(ph):
        den_q = dd[:, 2 * q] + dd[:, 2 * q + 1]
        x = (agg[:, q * H:(q + 1) * H] / (den_q[:, None] + 1e-16)
             + bb[q * H:(q + 1) * H][None, :])
        xs.append(x[None])
        s1 = jnp.sum(x, axis=0)[None, :]
        s2 = jnp.sum(x * x, axis=0)[None, :]
        news.append(jnp.concatenate([s1, s2, jnp.zeros((6, H), F32)],
                                    axis=0)[None])
    nm_ref[...] = jnp.concatenate(xs, axis=0)
    new = jnp.concatenate(news, axis=0)

    @pl.when(i == 0)
    def _init():
        st_ref[...] = new

    @pl.when(i > 0)
    def _acc():
        st_ref[...] = st_ref[...] + new


def _tc_combine(hd, ph, part, den4, b):
    np_ = hd // ph
    wp = ph * H
    return pl.pallas_call(
        functools.partial(_p1_body, ph),
        grid=(np_, NRB),
        in_specs=[
            pl.BlockSpec((2, 1, RB, wp), lambda p, i: (0, p, i, 0)),
            pl.BlockSpec((1, RB, 2 * ph), lambda p, i: (p, i, 0)),
            pl.BlockSpec((1, 1, wp), lambda p, i: (p, 0, 0)),
        ],
        out_specs=[
            pl.BlockSpec((ph, RB, H), lambda p, i: (p, i, 0)),
            pl.BlockSpec((ph, 8, H), lambda p, i: (p, 0, 0)),
        ],
        out_shape=[jax.ShapeDtypeStruct((hd, N, H), F32),
                   jax.ShapeDtypeStruct((hd, 8, H), F32)],
    )(part, den4, b.reshape(np_, 1, wp))


def _p2_body(hd, emit_table, nm_ref, st_ref, g_ref, be_ref, w2_ref, a2_ref,
             zt2_ref, alt2_ref):
    acc = jnp.zeros((RB, H), F32)
    for h in range(hd):
        x = nm_ref[h]
        mu = st_ref[h, 0] * (1.0 / N)
        var = st_ref[h, 1] * (1.0 / N) - mu * mu
        xb = jnp.maximum(
            g_ref[h][None, :] * (x - mu[None, :])
            / jnp.sqrt(var + 1e-5)[None, :] + be_ref[h][None, :], 0.0)
        acc = acc + _mm(xb, w2_ref[h])
    alt2_ref[...] = jnp.sum(acc * a2_ref[...], axis=1).reshape(RB, 1)
    if emit_table:
        zt2_ref[...] = acc


def _tc_bn_prep2(hd, emit_table, nm, st, g, be, w2r, a2):
    zt2, alt2 = pl.pallas_call(
        functools.partial(_p2_body, hd, emit_table),
        grid=(NRB,),
        in_specs=[
            pl.BlockSpec((hd, RB, H), lambda i: (0, i, 0)),
            pl.BlockSpec((hd, 8, H), lambda i: (0, 0, 0)),
            pl.BlockSpec((hd, H), lambda i: (0, 0)),
            pl.BlockSpec((hd, H), lambda i: (0, 0)),
            pl.BlockSpec((hd, H, H), lambda i: (0, 0, 0)),
            pl.BlockSpec((1, H), lambda i: (0, 0)),
        ],
        out_specs=[
            pl.BlockSpec((RB, H), lambda i: (i, 0)),
            pl.BlockSpec((RB, 1), lambda i: (i, 0)),
        ],
        out_shape=[jax.ShapeDtypeStruct((N, H), F32),
                   jax.ShapeDtypeStruct((N, 1), F32)],
    )(nm, st, g.reshape(hd, H), be.reshape(hd, H), w2r, a2)
    return zt2, alt2.reshape(-1)


def _final_body(part_ref, den_ref, b_ref, g_ref, be_ref,
                wc1_ref, bc1_ref, wc2_ref, bc2_ref, out_ref):
    agg = part_ref[0, 0] + part_ref[1, 0]
    den = den_ref[..., 0] + den_ref[..., 1]
    nc = agg / (den[:, None] + 1e-16) + b_ref[...][None, :]
    mu = jnp.mean(nc, axis=0)
    var = jnp.mean((nc - mu[None, :]) ** 2, axis=0)
    nc = jnp.maximum(
        g_ref[...][None, :] * (nc - mu[None, :])
        / jnp.sqrt(var + 1e-5)[None, :] + be_ref[...][None, :], 0.0)
    hh = jnp.maximum(
        _mm(nc, wc1_ref[...])
        + bc1_ref[...][None, :], 0.0)
    out_ref[...] = (_mm(hh, wc2_ref[...])
                    + bc2_ref[...][None, :])


def _tc_final(part, denT, b, g, be, wc1, bc1, wc2, bc2):
    return pl.pallas_call(
        _final_body,
        out_shape=jax.ShapeDtypeStruct((N, 2), F32),
    )(part, denT[:N], b, g, be, wc1, bc1, wc2, bc2)


def _sc_gat(hd, ph, sd, alsT, aldT, zsT, z2d, z1d):
    np_ = hd // ph
    wp = ph * H
    nz = -(-(hd * N + ph) // DZ)
    mden = nz * DZ
    mesh = plsc.VectorSubcoreMesh(core_axis_name="c", subcore_axis_name="s")

    @functools.partial(
        pl.kernel,
        out_type=[jax.ShapeDtypeStruct((2, np_, N, wp), F32),
                  jax.ShapeDtypeStruct((2 * mden,), F32)],
        mesh=mesh,
        scratch_types=[
            [pltpu.VMEM((2 * C,), jnp.int32) for _ in range(2)],
            [pltpu.VMEM((C,), jnp.int32) for _ in range(2)],
            [pltpu.VMEM((C,), jnp.int32) for _ in range(2)],
            [[pltpu.VMEM((C,), jnp.int32) for _ in range(ph)]
             for _ in range(2)],
            [[pltpu.VMEM((C,), jnp.int32) for _ in range(ph)]
             for _ in range(2)],
            [[pltpu.VMEM((C,), F32) for _ in range(ph)]
             for _ in range(2)],
            [[pltpu.VMEM((C,), F32) for _ in range(ph)]
             for _ in range(2)],
            [[pltpu.VMEM((C,), F32) for _ in range(ph)]
             for _ in range(2)],
            [pltpu.VMEM((C, wp), F32) for _ in range(2)],
            [pltpu.SemaphoreType.DMA for _ in range(2)],
            pltpu.VMEM_SHARED((N + 8, wp), F32),
            pltpu.VMEM_SHARED((mden,), F32),
        ],
        compiler_params=pltpu.CompilerParams(use_tc_tiling_on_sc=False),
    )
    def k(sd_hbm, als_hbm, ald_hbm, zst_hbm, z2d_hbm, z1d_hbm,
          part_hbm, den_hbm,
          sd_v, iz_v, d_v, ia_v, id_v, as_v, ad_v, ex_v, g_v, sem,
          agg_sh, den_sh):
        cid = lax.axis_index("c")
        sid = lax.axis_index("s")
        npair = jnp.where(cid == 0, NP0, NPAIR2 - NP0)
        wchunk = jnp.where(cid == 0, sid * (2 * NP0),
                           32 * NP0 + sid * (2 * (NPAIR2 - NP0)))

        @pl.when(sid < nz)
        def _zero_den():
            pltpu.sync_copy(z1d_hbm, den_sh.at[pl.ds(sid * DZ, DZ)])

        for p in range(np_):
            plsc.subcore_barrier()

            @pl.when(sid < 15)
            def _zero_big():
                pltpu.sync_copy(z2d_hbm,
                                agg_sh.at[pl.ds(sid * ZBIG, ZBIG), :])

            @pl.when(sid == 15)
            def _zero_last():
                pltpu.sync_copy(z2d_hbm.at[pl.ds(0, ZLAST + 8), :],
                                agg_sh.at[pl.ds(15 * ZBIG, ZLAST + 8), :])

            plsc.subcore_barrier()

            def issue(kb, b):
                base = (wchunk + kb) * (2 * C)
                pltpu.sync_copy(sd_hbm.at[pl.ds(base, 2 * C)], sd_v[b])
                for v in range(C // 16):
                    sl = pl.ds(v * 16, 16)
                    s16 = sd_v[b][sl] + (p * N)
                    d16 = sd_v[b][pl.ds(C + v * 16, 16)]
                    iz_v[b][sl] = s16
                    d_v[b][sl] = d16
                    dp = (d16 + (p * N)) * ph
                    sp = s16 * ph
                    for q in range(ph):
                        ia_v[b][q][sl] = sp + q
                        id_v[b][q][sl] = dp + q
                for q in range(ph):
                    pltpu.async_copy(als_hbm.at[ia_v[b][q]], as_v[b][q],
                                     sem[b])
                    pltpu.async_copy(ald_hbm.at[id_v[b][q]], ad_v[b][q],
                                     sem[b])
                pltpu.async_copy(zst_hbm.at[iz_v[b]], g_v[b], sem[b])

            def process(b):
                for q in range(ph):
                    pltpu.make_async_copy(als_hbm.at[ia_v[b][q]],
                                          as_v[b][q], sem[b]).wait()
                    pltpu.make_async_copy(ald_hbm.at[id_v[b][q]],
                                          ad_v[b][q], sem[b]).wait()
                pltpu.make_async_copy(zst_hbm.at[iz_v[b]], g_v[b],
                                      sem[b]).wait()
                for q in range(ph):
                    for v in range(C // 16):
                        sl = pl.ds(v * 16, 16)
                        x = as_v[b][q][sl] + ad_v[b][q][sl]
                        ex_v[b][q][sl] = jnp.exp(jnp.maximum(x, 0.2 * x))
                    pltpu.sync_copy(ex_v[b][q], den_sh.at[id_v[b][q]],
                                    add=True)

                def scale(g, c2):
                    exq = [ex_v[b][q][pl.ds(g * 16, 16)]
                           for q in range(ph)]
                    for l in range(16):
                        i = g * 16 + l
                        for q in range(ph):
                            sv = jnp.full((16,), exq[q][l], F32)
                            for j in range(H // 16):
                                slj = pl.ds((q * (H // 16) + j) * 16, 16)
                                g_v[b][i, slj] = g_v[b][i, slj] * sv
                    return c2

                lax.fori_loop(0, C // 16, scale, 0)
                pltpu.sync_copy(g_v[b], agg_sh.at[d_v[b]], add=True)

            issue(0, 0)

            def pair(pp, carry):
                k0 = 2 * pp
                issue(k0 + 1, 1)
                process(0)

                @pl.when(pp + 1 < npair)
                def _next():
                    issue(k0 + 2, 0)

                process(1)
                return carry

            lax.fori_loop(0, npair, pair, 0)
            plsc.subcore_barrier()

            @pl.when(sid < 15)
            def _wb_big():
                pltpu.sync_copy(
                    agg_sh.at[pl.ds(sid * ZBIG, ZBIG), :],
                    part_hbm.at[cid, p, pl.ds(sid * ZBIG, ZBIG), :])

            @pl.when(sid == 15)
            def _wb_last():
                pltpu.sync_copy(
                    agg_sh.at[pl.ds(15 * ZBIG, ZLAST), :],
                    part_hbm.at[cid, p, pl.ds(15 * ZBIG, ZLAST), :])

        plsc.subcore_barrier()

        @pl.when(sid == 0)
        def _write_den():
            pltpu.sync_copy(den_sh, den_hbm.at[pl.ds(cid * mden, mden)])

    part, den = k(sd, alsT, aldT, zsT, z2d, z1d)
    return part, den, mden


def kernel(x_card, x_merchant, edge_index_cm, edge_index_mc,
           W_ci, b_ci, W_mi, b_mi,
           W1_cm, as1_cm, ad1_cm, b1_cm, W1_mc, as1_mc, ad1_mc, b1_mc,
           g1_c, be1_c, g1_m, be1_m,
           W2_cm, as2_cm, ad2_cm, b2_cm, W2_mc, as2_mc, ad2_mc, b2_mc,
           g2_c, be2_c, g2_m, be2_m,
           Wc1, bc1, Wc2, bc2):
    ei_cm = edge_index_cm.astype(jnp.int32)
    ei_mc = edge_index_mc.astype(jnp.int32)
    pad_s = jnp.zeros((EP - E,), jnp.int32)
    pad_d = jnp.full((EP - E,), N, jnp.int32)

    def _pack(src, dst):
        s = jnp.concatenate([src, pad_s]).reshape(-1, C)
        d = jnp.concatenate([dst, pad_d]).reshape(-1, C)
        return jnp.concatenate([s, d], axis=1).reshape(-1)

    sd_cm = _pack(ei_cm[0], ei_cm[1])
    sd_mc = _pack(ei_mc[0], ei_mc[1])
    z2dw = jnp.zeros((ZBIG, 2 * H), F32)
    z2d1 = jnp.zeros((ZBIG, H), F32)
    z1d = jnp.zeros((DZ,), F32)
    w2r = W2_mc.reshape(H, 4, H).transpose(1, 0, 2)
    zp16 = jnp.zeros((16,), F32)

    xc, xm = _tc_proj(x_card, x_merchant, W_ci, b_ci, W_mi, b_mi)

    zsT_cm, alsT_cm, aldT_cm = _tc_prep(4, 2, xc, xm, W1_cm, as1_cm, ad1_cm)
    zsT_mc, alsT_mc, aldT_mc = _tc_prep(4, 2, xm, xc, W1_mc, as1_mc, ad1_mc)

    part_cm, den_cm, md1 = _sc_gat(4, 2, sd_cm, alsT_cm, aldT_cm, zsT_cm,
                                   z2dw, z1d)
    part_mc, den_mc, _ = _sc_gat(4, 2, sd_mc, alsT_mc, aldT_mc, zsT_mc,
                                 z2dw, z1d)

    def _den4(den):
        return jnp.stack([den[:4 * N], den[md1:md1 + 4 * N]],
                         axis=-1).reshape(2, N, 4)

    nm1, st_m = _tc_combine(4, 2, part_cm, _den4(den_cm), b1_cm)
    nc1, st_c = _tc_combine(4, 2, part_mc, _den4(den_mc), b1_mc)
    zsT2, als2 = _tc_bn_prep2(4, True, nm1, st_m, g1_m, be1_m, w2r, as2_mc)
    _, ald2 = _tc_bn_prep2(4, False, nc1, st_c, g1_c, be1_c, w2r, ad2_mc)
    als2 = jnp.concatenate([als2, zp16])
    ald2 = jnp.concatenate([ald2, zp16])

    part2, den2, md2 = _sc_gat(1, 1, sd_mc, als2, ald2, zsT2, z2d1, z1d)
    den2T = jnp.stack([den2[:N], den2[md2:md2 + N]], axis=-1)

    return _tc_final(part2, den2T, b2_mc, g2_c, be2_c, Wc1, bc1, Wc2, bc2)
